# baseline (speedup 1.0000x reference)
import functools

import jax
import jax.numpy as jnp
from jax import lax
from jax.experimental import pallas as pl
from jax.experimental.pallas import tpu as pltpu
from jax.experimental.pallas import tpu_sc as plsc

_HIDDEN = 64
_EPS = 1e-12
_NC = 2
_NS = 16
_NW = _NC * _NS
_BLK = 128
_SEQ = 50
_NBB = 16384 // _BLK
_NBLK = _SEQ * _NBB


def _ln_body(ids_hbm, table_hbm, gamma_hbm, beta_hbm, out_hbm,
             iv0, iv1, pidx0, pidx1, rows0, rows1, ov0, ov1, gb_v,
             isem, gsem, wsem):
    wid = lax.axis_index("s") * _NC + lax.axis_index("c")
    nper = _NBLK // _NW
    blk0 = wid * nper
    iv = (iv0, iv1)
    pidx = (pidx0, pidx1)
    rows = (rows0, rows1)
    ov = (ov0, ov1)

    pltpu.sync_copy(gamma_hbm, gb_v.at[0])
    pltpu.sync_copy(beta_hbm, gb_v.at[1])
    g4 = [gb_v[0, pl.ds(16 * j, 16)] for j in range(4)]
    b4 = [gb_v[1, pl.ds(16 * j, 16)] for j in range(4)]
    lanes = lax.iota(jnp.int32, 16)
    dnums = lax.GatherDimensionNumbers(
        offset_dims=(), collapsed_slice_dims=(0,), start_index_map=(0,))

    def _shuf(x, p):
        return lax.gather(x, p.reshape(16, 1), dnums, (1,),
                          indices_are_sorted=False, unique_indices=True,
                          mode=lax.GatherScatterMode.PROMISE_IN_BOUNDS)

    def _ids_src(i):
        blk = blk0 + i
        l = blk // _NBB
        bt = blk - l * _NBB
        return ids_hbm.at[l, pl.ds(bt * _BLK, _BLK)]

    def _out_dst(i):
        blk = blk0 + i
        l = blk // _NBB
        bt = blk - l * _NBB
        return out_hbm.at[l, :, bt, :, :]

    def _fire_idx(i, s):
        pltpu.async_copy(_ids_src(i), iv[s], isem.at[s])

    def _wait_idx(i, s):
        pltpu.make_async_copy(_ids_src(i), iv[s], isem.at[s]).wait()

    def _shift_fire_gather(s):
        for j in range(8):
            pidx[s][pl.ds(16 * j, 16)] = lax.shift_right_logical(
                iv[s][pl.ds(16 * j, 16)], 1)
        pltpu.async_copy(table_hbm.at[pidx[s]], rows[s], gsem.at[s])

    def _wait_gather(s):
        pltpu.make_async_copy(table_hbm.at[pidx[s]], rows[s],
                              gsem.at[s]).wait()

    def _fire_out(i, s):
        pltpu.async_copy(ov[s], _out_dst(i), wsem.at[s])

    def _wait_out(i, s):
        pltpu.make_async_copy(ov[s], _out_dst(i), wsem.at[s]).wait()

    def _compute(s):
        rv, o3 = rows[s], ov[s]

        def grp_body(gi, _):
            b0 = gi * 16
            rowv = lanes + b0
            iv16 = iv[s][pl.ds(b0, 16)]
            pv64 = lax.shift_left(lax.bitwise_and(iv16, 1), 6)
            acc = jnp.zeros((16,), jnp.float32)
            acq = jnp.zeros((16,), jnp.float32)
            cols = []
            for k in range(16):
                pk = lax.bitwise_and(lanes + k, 15)
                ck = pv64 + pk
                cols.append((pk, ck))
                for j in range(4):
                    x = plsc.load_gather(rv, [rowv, ck + 16 * j])
                    acc = acc + x
                    acq = acq + x * x
            mean = acc * (1.0 / 64.0)
            v = acq * (1.0 / 64.0) - mean * mean + _EPS
            ib = lax.bitcast_convert_type(v, jnp.int32)
            ib = jnp.int32(0x5F3759DF) - lax.shift_right_logical(ib, 1)
            y = lax.bitcast_convert_type(ib, jnp.float32)
            hv = 0.5 * v
            y = y * (1.5 - hv * y * y)
            y = y * (1.5 - hv * y * y)
            y = y * (1.5 - hv * y * y)
            for k in range(16):
                pk, ck = cols[k]
                d1 = lax.bitwise_and(pk, 7)
                for j in range(4):
                    grot = _shuf(g4[j], pk)
                    brot = _shuf(b4[j], pk)
                    x = plsc.load_gather(rv, [rowv, ck + 16 * j])
                    val = (x - mean) * y * grot + brot
                    hvv = pk + 16 * j
                    plsc.store_scatter(
                        o3, [lax.shift_right_logical(hvv, 3), d1, rowv], val)
            return 0

        lax.fori_loop(0, _BLK // 16, grp_body, 0)

    _fire_idx(0, 0)
    _wait_idx(0, 0)
    _shift_fire_gather(0)
    _fire_idx(1, 1)

    def step_body(st, _):
        for bi in range(2):
            i = st * 2 + bi
            s, o = bi, 1 - bi

            @pl.when(i + 1 < nper)
            def _():
                _wait_idx(i + 1, o)
                _shift_fire_gather(o)

            _wait_gather(s)

            @pl.when(i >= 2)
            def _():
                _wait_out(i - 2, s)

            _compute(s)

            @pl.when(i + 2 < nper)
            def _():
                _fire_idx(i + 2, s)

            _fire_out(i, s)
        return 0

    lax.fori_loop(0, nper // 2, step_body, 0)
    _wait_out(nper - 2, 0)
    _wait_out(nper - 1, 1)


def _make_call():
    mesh = plsc.VectorSubcoreMesh(core_axis_name="c", subcore_axis_name="s")
    return pl.kernel(
        _ln_body,
        out_type=jax.ShapeDtypeStruct((_SEQ, 8, _NBB, 8, _BLK), jnp.float32),
        mesh=mesh,
        scratch_types=[
            pltpu.VMEM((_BLK,), jnp.int32),
            pltpu.VMEM((_BLK,), jnp.int32),
            pltpu.VMEM((_BLK,), jnp.int32),
            pltpu.VMEM((_BLK,), jnp.int32),
            pltpu.VMEM((_BLK, 128), jnp.float32),
            pltpu.VMEM((_BLK, 128), jnp.float32),
            pltpu.VMEM((8, 8, _BLK), jnp.float32),
            pltpu.VMEM((8, 8, _BLK), jnp.float32),
            pltpu.VMEM((2, _HIDDEN), jnp.float32),
            pltpu.SemaphoreType.DMA((2,)),
            pltpu.SemaphoreType.DMA((2,)),
            pltpu.SemaphoreType.DMA((2,)),
        ],
        compiler_params=pltpu.CompilerParams(use_tc_tiling_on_sc=True,
                                             needs_layout_passes=False),
    )


@jax.jit
def kernel(idiom_ids, table, gamma, beta):
    ids_t = idiom_ids.T.astype(jnp.int32)
    table2 = table.reshape(500000, 128)
    out5 = _make_call()(ids_t, table2, gamma, beta)
    return out5.transpose(2, 4, 0, 1, 3).reshape(16384, _SEQ, _HIDDEN)
